# baseline (speedup 1.0000x reference)
import functools

import jax
import jax.numpy as jnp
from jax import lax
from jax.experimental import pallas as pl
from jax.experimental.pallas import tpu as pltpu
from jax.experimental.pallas import tpu_sc as plsc

N = 10000
E = 320000
D = 128

NC, NS, L = 2, 16, 16
NW = NC * NS
CHUNK = 112
CPT0 = 105
CPT1 = 75
EPT0 = CPT0 * CHUNK
EPT1 = (E - NS * EPT0) // NS
CPTA = 107
PCPT = 3
PPT = CHUNK * PCPT
NPAD = NW * PPT
NACC = 10240
RPS = NACC // NS
ZR = 80


def _mm2_body(x_ref, w1_ref, w2_ref, o1_ref, o2_ref):
    x = x_ref[...]
    o1_ref[...] = jnp.dot(x, w1_ref[...], preferred_element_type=jnp.float32)
    o2_ref[...] = jnp.dot(x, w2_ref[...], preferred_element_type=jnp.float32)


def _mm2(x, w1, w2):
    BM = 1000
    return pl.pallas_call(
        _mm2_body,
        grid=(N // BM,),
        in_specs=[pl.BlockSpec((BM, D), lambda i: (i, 0)),
                  pl.BlockSpec((D, D), lambda i: (0, 0)),
                  pl.BlockSpec((D, D), lambda i: (0, 0))],
        out_specs=[pl.BlockSpec((BM, D), lambda i: (i, 0)),
                   pl.BlockSpec((BM, D), lambda i: (i, 0))],
        out_shape=[jax.ShapeDtypeStruct((N, D), jnp.float32),
                   jax.ShapeDtypeStruct((N, D), jnp.float32)],
    )(x, w1, w2)


_MESH = plsc.VectorSubcoreMesh(core_axis_name="c", subcore_axis_name="s")


@functools.partial(
    pl.kernel,
    out_type=[
        jax.ShapeDtypeStruct((NC, NACC, D), jnp.float32),
        jax.ShapeDtypeStruct((NPAD, D), jnp.float32),
        jax.ShapeDtypeStruct((NPAD, D), jnp.float32),
    ],
    mesh=_MESH,
    scratch_types=[
        pltpu.VMEM((2, CHUNK), jnp.int32),
        pltpu.VMEM((2, CHUNK), jnp.int32),
        pltpu.VMEM((2, CHUNK), jnp.int32),
        pltpu.VMEM((1, CHUNK), jnp.float32),
        pltpu.VMEM((1, CHUNK), jnp.float32),
        pltpu.VMEM((1, CHUNK), jnp.float32),
        pltpu.VMEM((CHUNK, D), jnp.float32),
        pltpu.VMEM((CHUNK, D), jnp.float32),
        pltpu.VMEM((CHUNK, D), jnp.float32),
        pltpu.VMEM((PCPT, CHUNK), jnp.int32),
        pltpu.VMEM_SHARED((NACC, D), jnp.float32),
        pltpu.SemaphoreType.DMA,
        pltpu.SemaphoreType.DMA,
        pltpu.SemaphoreType.DMA,
        pltpu.SemaphoreType.DMA,
        pltpu.SemaphoreType.DMA,
        pltpu.SemaphoreType.DMA,
        pltpu.SemaphoreType.DMA,
        pltpu.SemaphoreType.DMA,
        pltpu.SemaphoreType.DMA,
    ],
)
def _sc_spmm(h_hbm, re_hbm, g0_hbm, ecv_hbm, vals_hbm, perm_hbm,
             part_hbm, left_hbm, g0p_hbm,
             ech0, ech1, ech2, vch0, vch1, vch2, gb0, gb1, gb2, pidx_v, acc,
             semi0, semi1, semi2, semg0, semg1, semg2, sema0, sema1, sema2):
    c = lax.axis_index("c")
    s = lax.axis_index("s")
    wid = s * NC + c
    ech, vch, gb = (ech0, ech1, ech2), (vch0, vch1, vch2), (gb0, gb1, gb2)
    semi, semg, sema = (semi0, semi1, semi2), (semg0, semg1, semg2), (sema0, sema1, sema2)

    def _zrow(i, carry):
        for q in range(D // L):
            gb2[i, pl.ds(q * L, L)] = jnp.zeros((L,), jnp.float32)
        return carry
    lax.fori_loop(0, CHUNK, _zrow, 0)
    zsrc = gb2.at[pl.ds(0, ZR)]
    for k in range(RPS // ZR):
        pltpu.sync_copy(zsrc, acc.at[pl.ds(s * RPS + k * ZR, ZR)])

    pltpu.sync_copy(perm_hbm.at[wid], pidx_v)
    pseq = [(re_hbm, left_hbm, k) for k in range(PCPT)] + \
           [(g0_hbm, g0p_hbm, k) for k in range(PCPT)]

    def _pstart(i):
        src, _, k = pseq[i]
        pltpu.async_copy(src.at[pidx_v.at[k]], gb[i % 2], semg[i % 2])

    _pstart(0)
    for i in range(len(pseq)):
        if i + 1 < len(pseq):
            _pstart(i + 1)
        src, dst, k = pseq[i]
        pltpu.make_async_copy(src.at[pidx_v.at[k]], gb[i % 2], semg[i % 2]).wait()
        pltpu.sync_copy(gb[i % 2], dst.at[pl.ds(wid * PPT + k * CHUNK, CHUNK)])

    plsc.subcore_barrier()

    def _fetch(j, b):
        pltpu.async_copy(ecv_hbm.at[wid, j], ech[b], semi[b])
        pltpu.async_copy(vals_hbm.at[wid, pl.ds(j, 1)], vch[b], semi[b])

    def _wait_fetch(j, b):
        pltpu.make_async_copy(ecv_hbm.at[wid, j], ech[b], semi[b]).wait()
        pltpu.make_async_copy(vals_hbm.at[wid, pl.ds(j, 1)], vch[b], semi[b]).wait()

    def _gather(j, b):
        _wait_fetch(j, b)
        pltpu.async_copy(h_hbm.at[ech[b].at[0]], gb[b], semg[b])

    def _process(j, b):
        pltpu.make_async_copy(h_hbm.at[ech[b].at[0]], gb[b], semg[b]).wait()

        def _group(g, carry2):
            vv = vch[b][0, pl.ds(g * L, L)]
            for e in range(L):
                val = vv[e]
                r = g * L + e
                for q in range(D // L):
                    gb[b][r, pl.ds(q * L, L)] = gb[b][r, pl.ds(q * L, L)] * val
            return carry2
        lax.fori_loop(0, CHUNK // L, _group, 0)
        pltpu.async_copy(gb[b], acc.at[ech[b].at[1]], sema[b], add=True)

    def _drain(b):
        pltpu.make_async_copy(gb[b], acc.at[ech[b].at[1]], sema[b]).wait()

    def _run_edges(cpt):
        _fetch(CPTA - 1, 2)
        _wait_fetch(CPTA - 1, 2)
        pltpu.async_copy(gb2, acc.at[ech2.at[1]], sema2, add=True)
        _fetch(0, 0)
        _gather(0, 0)
        _fetch(1, 1)

        def _triple(t, carry):
            for b in range(3):
                j = 3 * t + b
                _gather(j + 1, (b + 1) % 3)
                _process(j, b)
                _drain((b + 2) % 3)
                _fetch(j + 2, (b + 2) % 3)
            return carry
        lax.fori_loop(0, cpt // 3, _triple, 0)

        _drain(2)
        pltpu.make_async_copy(h_hbm.at[ech[0].at[0]], gb[0], semg[0]).wait()
        _wait_fetch(cpt + 1, 1)

    @pl.when(c == 0)
    def _edges_c0():
        _run_edges(CPT0)

    @pl.when(c != 0)
    def _edges_c1():
        _run_edges(CPT1)

    plsc.subcore_barrier()

    for k in range(RPS // ZR):
        r0 = s * RPS + k * ZR
        pltpu.sync_copy(acc.at[pl.ds(r0, ZR)], part_hbm.at[c, pl.ds(r0, ZR)])


def _fuse_body(p_ref, left_ref, g0p_ref, b_ref, o_ref):
    gate = jax.nn.sigmoid(g0p_ref[...] + b_ref[...])
    agg = jnp.maximum(p_ref[0] + p_ref[1], 0.0)
    o_ref[...] = gate * agg + (1.0 - gate) * left_ref[...]


def _fuse(part, left, g0p, b):
    BM = 1000
    return pl.pallas_call(
        _fuse_body,
        grid=(N // BM,),
        in_specs=[pl.BlockSpec((NC, BM, D), lambda i: (0, i, 0)),
                  pl.BlockSpec((BM, D), lambda i: (i, 0)),
                  pl.BlockSpec((BM, D), lambda i: (i, 0)),
                  pl.BlockSpec((1, D), lambda i: (0, 0))],
        out_specs=pl.BlockSpec((BM, D), lambda i: (i, 0)),
        out_shape=jax.ShapeDtypeStruct((N, D), jnp.float32),
    )(part, left, g0p, b)


def kernel(right_embed, edge_index, adj_vals, perm, gcnW1, highwayWr, highwaybr):
    right_embed = right_embed.astype(jnp.float32)
    h, g0 = _mm2(right_embed, gcnW1.astype(jnp.float32),
                 highwayWr.astype(jnp.float32))

    tpc = CPTA * CHUNK

    def shard(flat):
        g0 = jnp.pad(flat[:NS * EPT0].reshape(NS, EPT0),
                     ((0, 0), (0, tpc - EPT0)))
        g1 = jnp.pad(flat[NS * EPT0:].reshape(NS, EPT1),
                     ((0, 0), (0, tpc - EPT1)))
        return jnp.stack([g0, g1], axis=1).reshape(NW, CPTA, CHUNK)

    rows3 = shard(edge_index[0]).astype(jnp.int32)
    cols3 = shard(edge_index[1]).astype(jnp.int32)
    vals3 = shard(adj_vals.astype(jnp.float32))
    ecv = jnp.stack([cols3, rows3], axis=2)
    perm3 = jnp.pad(perm, (0, NPAD - N)).reshape(NW, PCPT, CHUNK).astype(jnp.int32)

    part, left_pad, g0p_pad = _sc_spmm(h, right_embed, g0, ecv, vals3, perm3)

    return _fuse(part[:, :N], left_pad[:N], g0p_pad[:N],
                 highwaybr.astype(jnp.float32).reshape(1, D))


# speedup vs baseline: 1.0113x; 1.0113x over previous
import functools

import jax
import jax.numpy as jnp
from jax import lax
from jax.experimental import pallas as pl
from jax.experimental.pallas import tpu as pltpu
from jax.experimental.pallas import tpu_sc as plsc

N = 10000
E = 320000
D = 128

NC, NS, L = 2, 16, 16
NW = NC * NS
CHUNK = 112
CPT0 = 75
CPT1 = 105
EPT0 = CPT0 * CHUNK
EPT1 = (E - NS * EPT0) // NS
CPTA = 107
PCPT = 3
PPT = CHUNK * PCPT
NPAD = NW * PPT
NACC = 10240
RPS = NACC // NS
ZR = 80


def _mm2_body(x_ref, w1_ref, w2_ref, o1_ref, o2_ref):
    x = x_ref[...]
    o1_ref[...] = jnp.dot(x, w1_ref[...], preferred_element_type=jnp.float32)
    o2_ref[...] = jnp.dot(x, w2_ref[...], preferred_element_type=jnp.float32)


def _mm2(x, w1, w2):
    BM = 1000
    return pl.pallas_call(
        _mm2_body,
        grid=(N // BM,),
        in_specs=[pl.BlockSpec((BM, D), lambda i: (i, 0)),
                  pl.BlockSpec((D, D), lambda i: (0, 0)),
                  pl.BlockSpec((D, D), lambda i: (0, 0))],
        out_specs=[pl.BlockSpec((BM, D), lambda i: (i, 0)),
                   pl.BlockSpec((BM, D), lambda i: (i, 0))],
        out_shape=[jax.ShapeDtypeStruct((N, D), jnp.float32),
                   jax.ShapeDtypeStruct((N, D), jnp.float32)],
    )(x, w1, w2)


_MESH = plsc.VectorSubcoreMesh(core_axis_name="c", subcore_axis_name="s")


@functools.partial(
    pl.kernel,
    out_type=[
        jax.ShapeDtypeStruct((NC, NACC, D), jnp.float32),
        jax.ShapeDtypeStruct((NPAD, D), jnp.float32),
        jax.ShapeDtypeStruct((NPAD, D), jnp.float32),
    ],
    mesh=_MESH,
    scratch_types=[
        pltpu.VMEM((2, CHUNK), jnp.int32),
        pltpu.VMEM((2, CHUNK), jnp.int32),
        pltpu.VMEM((2, CHUNK), jnp.int32),
        pltpu.VMEM((1, CHUNK), jnp.float32),
        pltpu.VMEM((1, CHUNK), jnp.float32),
        pltpu.VMEM((1, CHUNK), jnp.float32),
        pltpu.VMEM((CHUNK, D), jnp.float32),
        pltpu.VMEM((CHUNK, D), jnp.float32),
        pltpu.VMEM((CHUNK, D), jnp.float32),
        pltpu.VMEM((PCPT, CHUNK), jnp.int32),
        pltpu.VMEM_SHARED((NACC, D), jnp.float32),
        pltpu.SemaphoreType.DMA,
        pltpu.SemaphoreType.DMA,
        pltpu.SemaphoreType.DMA,
        pltpu.SemaphoreType.DMA,
        pltpu.SemaphoreType.DMA,
        pltpu.SemaphoreType.DMA,
        pltpu.SemaphoreType.DMA,
        pltpu.SemaphoreType.DMA,
        pltpu.SemaphoreType.DMA,
    ],
)
def _sc_spmm(h_hbm, re_hbm, g0_hbm, ecv_hbm, vals_hbm, perm_hbm,
             part_hbm, left_hbm, g0p_hbm,
             ech0, ech1, ech2, vch0, vch1, vch2, gb0, gb1, gb2, pidx_v, acc,
             semi0, semi1, semi2, semg0, semg1, semg2, sema0, sema1, sema2):
    c = lax.axis_index("c")
    s = lax.axis_index("s")
    wid = s * NC + c
    ech, vch, gb = (ech0, ech1, ech2), (vch0, vch1, vch2), (gb0, gb1, gb2)
    semi, semg, sema = (semi0, semi1, semi2), (semg0, semg1, semg2), (sema0, sema1, sema2)

    def _zrow(i, carry):
        for q in range(D // L):
            gb2[i, pl.ds(q * L, L)] = jnp.zeros((L,), jnp.float32)
        return carry
    lax.fori_loop(0, CHUNK, _zrow, 0)
    zsrc = gb2.at[pl.ds(0, ZR)]
    for k in range(RPS // ZR):
        pltpu.sync_copy(zsrc, acc.at[pl.ds(s * RPS + k * ZR, ZR)])

    pltpu.sync_copy(perm_hbm.at[wid], pidx_v)
    pseq = [(re_hbm, left_hbm, k) for k in range(PCPT)] + \
           [(g0_hbm, g0p_hbm, k) for k in range(PCPT)]

    def _pstart(i):
        src, _, k = pseq[i]
        pltpu.async_copy(src.at[pidx_v.at[k]], gb[i % 2], semg[i % 2])

    _pstart(0)
    for i in range(len(pseq)):
        if i + 1 < len(pseq):
            _pstart(i + 1)
        src, dst, k = pseq[i]
        pltpu.make_async_copy(src.at[pidx_v.at[k]], gb[i % 2], semg[i % 2]).wait()
        pltpu.sync_copy(gb[i % 2], dst.at[pl.ds(wid * PPT + k * CHUNK, CHUNK)])

    plsc.subcore_barrier()

    def _fetch(j, b):
        pltpu.async_copy(ecv_hbm.at[wid, j], ech[b], semi[b])
        pltpu.async_copy(vals_hbm.at[wid, pl.ds(j, 1)], vch[b], semi[b])

    def _wait_fetch(j, b):
        pltpu.make_async_copy(ecv_hbm.at[wid, j], ech[b], semi[b]).wait()
        pltpu.make_async_copy(vals_hbm.at[wid, pl.ds(j, 1)], vch[b], semi[b]).wait()

    def _gather(j, b):
        _wait_fetch(j, b)
        pltpu.async_copy(h_hbm.at[ech[b].at[0]], gb[b], semg[b])

    def _process(j, b):
        pltpu.make_async_copy(h_hbm.at[ech[b].at[0]], gb[b], semg[b]).wait()

        def _group(g, carry2):
            vv = vch[b][0, pl.ds(g * L, L)]
            for e in range(L):
                val = vv[e]
                r = g * L + e
                for q in range(D // L):
                    gb[b][r, pl.ds(q * L, L)] = gb[b][r, pl.ds(q * L, L)] * val
            return carry2
        lax.fori_loop(0, CHUNK // L, _group, 0)
        pltpu.async_copy(gb[b], acc.at[ech[b].at[1]], sema[b], add=True)

    def _drain(b):
        pltpu.make_async_copy(gb[b], acc.at[ech[b].at[1]], sema[b]).wait()

    def _run_edges(cpt):
        _fetch(CPTA - 1, 2)
        _wait_fetch(CPTA - 1, 2)
        pltpu.async_copy(gb2, acc.at[ech2.at[1]], sema2, add=True)
        _fetch(0, 0)
        _gather(0, 0)
        _fetch(1, 1)

        def _triple(t, carry):
            for b in range(3):
                j = 3 * t + b
                _gather(j + 1, (b + 1) % 3)
                _process(j, b)
                _drain((b + 2) % 3)
                _fetch(j + 2, (b + 2) % 3)
            return carry
        lax.fori_loop(0, cpt // 3, _triple, 0)

        _drain(2)
        pltpu.make_async_copy(h_hbm.at[ech[0].at[0]], gb[0], semg[0]).wait()
        _wait_fetch(cpt + 1, 1)

    @pl.when(c == 0)
    def _edges_c0():
        _run_edges(CPT0)

    @pl.when(c != 0)
    def _edges_c1():
        _run_edges(CPT1)

    plsc.subcore_barrier()

    for k in range(RPS // ZR):
        r0 = s * RPS + k * ZR
        pltpu.sync_copy(acc.at[pl.ds(r0, ZR)], part_hbm.at[c, pl.ds(r0, ZR)])


def _fuse_body(p_ref, left_ref, g0p_ref, b_ref, o_ref):
    gate = jax.nn.sigmoid(g0p_ref[...] + b_ref[...])
    agg = jnp.maximum(p_ref[0] + p_ref[1], 0.0)
    o_ref[...] = gate * agg + (1.0 - gate) * left_ref[...]


def _fuse(part, left, g0p, b):
    BM = 1000
    return pl.pallas_call(
        _fuse_body,
        grid=(N // BM,),
        in_specs=[pl.BlockSpec((NC, BM, D), lambda i: (0, i, 0)),
                  pl.BlockSpec((BM, D), lambda i: (i, 0)),
                  pl.BlockSpec((BM, D), lambda i: (i, 0)),
                  pl.BlockSpec((1, D), lambda i: (0, 0))],
        out_specs=pl.BlockSpec((BM, D), lambda i: (i, 0)),
        out_shape=jax.ShapeDtypeStruct((N, D), jnp.float32),
    )(part, left, g0p, b)


def kernel(right_embed, edge_index, adj_vals, perm, gcnW1, highwayWr, highwaybr):
    right_embed = right_embed.astype(jnp.float32)
    h, g0 = _mm2(right_embed, gcnW1.astype(jnp.float32),
                 highwayWr.astype(jnp.float32))

    tpc = CPTA * CHUNK

    def shard(flat):
        g0 = jnp.pad(flat[:NS * EPT0].reshape(NS, EPT0),
                     ((0, 0), (0, tpc - EPT0)))
        g1 = jnp.pad(flat[NS * EPT0:].reshape(NS, EPT1),
                     ((0, 0), (0, tpc - EPT1)))
        return jnp.stack([g0, g1], axis=1).reshape(NW, CPTA, CHUNK)

    rows3 = shard(edge_index[0]).astype(jnp.int32)
    cols3 = shard(edge_index[1]).astype(jnp.int32)
    vals3 = shard(adj_vals.astype(jnp.float32))
    ecv = jnp.stack([cols3, rows3], axis=2)
    perm3 = jnp.pad(perm, (0, NPAD - N)).reshape(NW, PCPT, CHUNK).astype(jnp.int32)

    part, left_pad, g0p_pad = _sc_spmm(h, right_embed, g0, ecv, vals3, perm3)

    return _fuse(part[:, :N], left_pad[:N], g0p_pad[:N],
                 highwaybr.astype(jnp.float32).reshape(1, D))
